# baseline (device time: 46094 ns/iter reference)
import jax
import jax.numpy as jnp
from jax import lax
from jax.experimental import pallas as pl
from jax.experimental.pallas import tpu as pltpu

N_DEV = 4
SQ = 256
SKV = 4096
HL = 8
DH = 128
DM = 1024
QB = 64
NC = 4
KPC = SKV // NC
NSB = KPC // QB
SCALE = 0.08838834764831843


def kernel(x, Wq, K_ext, V_ext, Wo):
    def body(x_hbm, wq_hbm, k_hbm, v_hbm, wo_hbm, out_ref,
             xv, wqv, wov, kbuf, vbuf, comm, in_sems, kv_sems,
             send_sems, recv_sems):
        p = lax.axis_index("i")
        peers = [lax.rem(p + d, N_DEV) for d in range(1, N_DEV)]

        def in_copies():
            return [
                pltpu.make_async_copy(x_hbm.at[0], xv, in_sems.at[0]),
                pltpu.make_async_copy(wq_hbm, wqv, in_sems.at[1]),
                pltpu.make_async_copy(wo_hbm, wov, in_sems.at[2]),
            ]

        for cp in in_copies():
            cp.start()

        def kv_copies():
            copies = []
            for c in range(NC):
                for sb in range(NSB):
                    row0 = QB * (NC * sb + c)
                    copies.append(pltpu.make_async_copy(
                        k_hbm.at[0, pl.ds(row0, QB), pl.ds(HL * p, HL), :],
                        kbuf.at[c, pl.ds(QB * sb, QB), :, :],
                        kv_sems.at[0, c]))
                    copies.append(pltpu.make_async_copy(
                        v_hbm.at[0, pl.ds(row0, QB), pl.ds(HL * p, HL), :],
                        vbuf.at[c, pl.ds(QB * sb, QB), :, :],
                        kv_sems.at[1, c]))
            return copies

        for cp in kv_copies():
            cp.start()

        barrier_sem = pltpu.get_barrier_semaphore()
        for nbr in peers:
            pl.semaphore_signal(
                barrier_sem, inc=1,
                device_id=(nbr,), device_id_type=pl.DeviceIdType.MESH)
        pl.semaphore_wait(barrier_sem, N_DEV - 1)

        xcp, wqcp, wocp = in_copies()
        xcp.wait()
        wqcp.wait()
        q = jnp.dot(xv[:, :], wqv[:, :],
                    preferred_element_type=jnp.float32) * SCALE

        rdmas = {}

        def broadcast(c):
            for d in range(1, N_DEV):
                r = pltpu.make_async_remote_copy(
                    src_ref=comm.at[c, 0],
                    dst_ref=comm.at[c, d],
                    send_sem=send_sems.at[c, d - 1],
                    recv_sem=recv_sems.at[c, d - 1],
                    device_id=(peers[d - 1],),
                    device_id_type=pl.DeviceIdType.MESH)
                rdmas[(c, d)] = r
                r.start()

        waiters = kv_copies()
        for c in range(NC):
            for cp in waiters[2 * NSB * c:2 * NSB * (c + 1)]:
                cp.wait()
            qc = q[QB * c:QB * (c + 1), :]
            ctx_parts = []
            for h in range(HL):
                qh = qc[:, h * DH:(h + 1) * DH]
                kh = kbuf[c, :, h, :]
                vh = vbuf[c, :, h, :]
                s = lax.dot_general(
                    qh, kh, (((1,), (1,)), ((), ())),
                    preferred_element_type=jnp.float32)
                w = jnp.exp(s)
                d = jnp.sum(w, axis=1, keepdims=True)
                ctx_parts.append(
                    jnp.dot(w, vh, preferred_element_type=jnp.float32) / d)
            ctx_c = jnp.concatenate(ctx_parts, axis=1)
            if c == 0:
                wocp.wait()
            comm[c, 0] = jnp.dot(
                ctx_c, wov[:, :],
                preferred_element_type=jnp.float32).astype(jnp.bfloat16)
            broadcast(c)

        for c in range(NC):
            for d in range(1, N_DEV):
                rdmas[(c, d)].wait_recv()
            out_ref[0, QB * c:QB * (c + 1), :] = (
                (comm[c, 0].astype(jnp.float32) +
                 comm[c, 1].astype(jnp.float32)) +
                (comm[c, 2].astype(jnp.float32) +
                 comm[c, 3].astype(jnp.float32)))
        for r in rdmas.values():
            r.wait_send()

    return pl.pallas_call(
        body,
        out_shape=jax.ShapeDtypeStruct((1, SQ, DM), jnp.float32),
        in_specs=[
            pl.BlockSpec(memory_space=pl.ANY),
            pl.BlockSpec(memory_space=pl.ANY),
            pl.BlockSpec(memory_space=pl.ANY),
            pl.BlockSpec(memory_space=pl.ANY),
            pl.BlockSpec(memory_space=pl.ANY),
        ],
        out_specs=pl.BlockSpec(memory_space=pltpu.VMEM),
        scratch_shapes=[
            pltpu.VMEM((SQ, DM), jnp.float32),
            pltpu.VMEM((DM, DM), jnp.float32),
            pltpu.VMEM((DM, DM), jnp.float32),
            pltpu.VMEM((NC, KPC, HL, DH), jnp.float32),
            pltpu.VMEM((NC, KPC, HL, DH), jnp.float32),
            pltpu.VMEM((NC, N_DEV, QB, DM), jnp.bfloat16),
            pltpu.SemaphoreType.DMA((3,)),
            pltpu.SemaphoreType.DMA((2, NC)),
            pltpu.SemaphoreType.DMA((NC, N_DEV - 1)),
            pltpu.SemaphoreType.DMA((NC, N_DEV - 1)),
        ],
        compiler_params=pltpu.CompilerParams(
            collective_id=0,
            vmem_limit_bytes=60 * 1024 * 1024,
        ),
    )(x, Wq, K_ext, V_ext, Wo)


# device time: 40192 ns/iter; 1.1468x vs baseline; 1.1468x over previous
import jax
import jax.numpy as jnp
from jax import lax
from jax.experimental import pallas as pl
from jax.experimental.pallas import tpu as pltpu

N_DEV = 4
SQ = 256
SKV = 4096
HL = 8
DH = 128
DM = 1024
QB = 64
NC = 4
KPC = SKV // NC
NSB = KPC // QB
SCALE = 0.08838834764831843


def kernel(x, Wq, K_ext, V_ext, Wo):
    def body(x_ref, wq_ref, k_hbm, v_hbm, wo_ref, out_ref,
             kbuf, vbuf, comm, kv_sems, send_sems, recv_sems):
        p = lax.axis_index("i")
        peers = [lax.rem(p + d, N_DEV) for d in range(1, N_DEV)]

        def kv_copies():
            copies = []
            for c in range(NC):
                for sb in range(NSB):
                    row0 = QB * (NC * sb + c)
                    copies.append(pltpu.make_async_copy(
                        k_hbm.at[0, pl.ds(row0, QB), pl.ds(HL * p, HL), :],
                        kbuf.at[c, pl.ds(QB * sb, QB), :, :],
                        kv_sems.at[0, c]))
                    copies.append(pltpu.make_async_copy(
                        v_hbm.at[0, pl.ds(row0, QB), pl.ds(HL * p, HL), :],
                        vbuf.at[c, pl.ds(QB * sb, QB), :, :],
                        kv_sems.at[1, c]))
            return copies

        for cp in kv_copies():
            cp.start()

        barrier_sem = pltpu.get_barrier_semaphore()
        for nbr in peers:
            pl.semaphore_signal(
                barrier_sem, inc=1,
                device_id=(nbr,), device_id_type=pl.DeviceIdType.MESH)
        pl.semaphore_wait(barrier_sem, N_DEV - 1)

        q = jnp.dot(x_ref[0], wq_ref[:, :],
                    preferred_element_type=jnp.float32) * SCALE

        rdmas = {}

        def broadcast(c):
            for d in range(1, N_DEV):
                r = pltpu.make_async_remote_copy(
                    src_ref=comm.at[c, 0],
                    dst_ref=comm.at[c, d],
                    send_sem=send_sems.at[c, d - 1],
                    recv_sem=recv_sems.at[c, d - 1],
                    device_id=(peers[d - 1],),
                    device_id_type=pl.DeviceIdType.MESH)
                rdmas[(c, d)] = r
                r.start()

        waiters = kv_copies()
        for c in range(NC):
            for cp in waiters[2 * NSB * c:2 * NSB * (c + 1)]:
                cp.wait()
            qc = q[QB * c:QB * (c + 1), :]
            ctx_parts = []
            for h in range(HL):
                qh = qc[:, h * DH:(h + 1) * DH]
                kh = kbuf[c, :, h, :]
                vh = vbuf[c, :, h, :]
                s = lax.dot_general(
                    qh, kh, (((1,), (1,)), ((), ())),
                    preferred_element_type=jnp.float32)
                w = jnp.exp(s)
                d = jnp.sum(w, axis=1, keepdims=True)
                ctx_parts.append(
                    jnp.dot(w, vh, preferred_element_type=jnp.float32) / d)
            ctx_c = jnp.concatenate(ctx_parts, axis=1)
            comm[c, 0] = jnp.dot(
                ctx_c, wo_ref[:, :],
                preferred_element_type=jnp.float32).astype(jnp.bfloat16)
            broadcast(c)

        for c in range(NC):
            for d in range(1, N_DEV):
                rdmas[(c, d)].wait_recv()
            out_ref[0, QB * c:QB * (c + 1), :] = (
                (comm[c, 0].astype(jnp.float32) +
                 comm[c, 1].astype(jnp.float32)) +
                (comm[c, 2].astype(jnp.float32) +
                 comm[c, 3].astype(jnp.float32)))
        for r in rdmas.values():
            r.wait_send()

    return pl.pallas_call(
        body,
        out_shape=jax.ShapeDtypeStruct((1, SQ, DM), jnp.float32),
        in_specs=[
            pl.BlockSpec(memory_space=pltpu.VMEM),
            pl.BlockSpec(memory_space=pltpu.VMEM),
            pl.BlockSpec(memory_space=pl.ANY),
            pl.BlockSpec(memory_space=pl.ANY),
            pl.BlockSpec(memory_space=pltpu.VMEM),
        ],
        out_specs=pl.BlockSpec(memory_space=pltpu.VMEM),
        scratch_shapes=[
            pltpu.VMEM((NC, KPC, HL, DH), jnp.float32),
            pltpu.VMEM((NC, KPC, HL, DH), jnp.float32),
            pltpu.VMEM((NC, N_DEV, QB, DM), jnp.bfloat16),
            pltpu.SemaphoreType.DMA((2, NC)),
            pltpu.SemaphoreType.DMA((NC, N_DEV - 1)),
            pltpu.SemaphoreType.DMA((NC, N_DEV - 1)),
        ],
        compiler_params=pltpu.CompilerParams(
            collective_id=0,
            vmem_limit_bytes=60 * 1024 * 1024,
        ),
    )(x, Wq, K_ext, V_ext, Wo)
